# baseline (speedup 1.0000x reference)
import jax
import jax.numpy as jnp
from jax import lax
from jax.experimental import pallas as pl
from jax.experimental.pallas import tpu as pltpu
from jax.experimental.pallas import tpu_sc as plsc

VOCAB = 100000
EMB_DIM = 128
BATCH = 4096
HIST = 50

NUM_CORES = 2
NUM_SUBCORES = 16
NUM_WORKERS = NUM_CORES * NUM_SUBCORES
NSPLIT = 2
SPLIT_BATCH = BATCH // NSPLIT
BATCH_PER_WORKER = SPLIT_BATCH // NUM_WORKERS
SUPER = 2
NSUP = BATCH_PER_WORKER // SUPER
NBUF = 8
LOOK = NBUF - 1


def _emb_body(idx_hbm, table_hbm, out_hbm, idx_v, *bufs_and_sems):
    bufs = bufs_and_sems[:NBUF]
    gsems = bufs_and_sems[NBUF:2 * NBUF]
    ssems = bufs_and_sems[2 * NBUF:3 * NBUF]

    wid = lax.axis_index("s") * NUM_CORES + lax.axis_index("c")
    pltpu.sync_copy(idx_hbm.at[wid], idx_v)
    base = wid * BATCH_PER_WORKER

    def fire_gather(c, b):
        for k in range(SUPER):
            pltpu.async_copy(
                table_hbm.at[idx_v.at[c * SUPER + k]],
                bufs[b].at[k], gsems[b])

    def wait_gather(c, b):
        for k in range(SUPER):
            pltpu.make_async_copy(
                table_hbm.at[idx_v.at[0]], bufs[b].at[k], gsems[b]).wait()

    def fire_scatter(c, b):
        pltpu.async_copy(
            bufs[b], out_hbm.at[pl.ds(base + c * SUPER, SUPER)], ssems[b])

    def wait_scatter(b):
        pltpu.make_async_copy(
            bufs[b], out_hbm.at[pl.ds(base, SUPER)], ssems[b]).wait()

    for b in range(LOOK):
        fire_gather(b, b)

    wait_gather(0, 0)
    fire_scatter(0, 0)
    fire_gather(LOOK, LOOK % NBUF)

    def step(c, b, tb):
        wait_scatter(tb)
        fire_gather(c + LOOK, tb)
        wait_gather(c, b)
        fire_scatter(c, b)

    nsteady = NSUP - LOOK - 1
    ngroups = nsteady // NBUF
    nrem = nsteady % NBUF

    def outer(g, carry):
        for bp in range(NBUF):
            c = g * NBUF + 1 + bp
            step(c, (bp + 1) % NBUF, (1 + bp + LOOK) % NBUF)
        return carry

    if ngroups > 0:
        lax.fori_loop(0, ngroups, outer, 0)
    for r in range(nrem):
        c = ngroups * NBUF + 1 + r
        step(c, c % NBUF, (c + LOOK) % NBUF)

    for c in range(NSUP - LOOK, NSUP):
        b = c % NBUF
        wait_gather(c, b)
        fire_scatter(c, b)
    for b in range(NBUF):
        wait_scatter(b)


def _emb_call(idx, weight):
    mesh = plsc.VectorSubcoreMesh(
        core_axis_name="c", subcore_axis_name="s",
        num_cores=NUM_CORES, num_subcores=NUM_SUBCORES,
    )
    run = pl.kernel(
        _emb_body,
        out_type=jax.ShapeDtypeStruct((SPLIT_BATCH, HIST, EMB_DIM), jnp.float32),
        mesh=mesh,
        scratch_types=(
            [pltpu.VMEM((BATCH_PER_WORKER, HIST), jnp.int32)]
            + [pltpu.VMEM((SUPER, HIST, EMB_DIM), jnp.float32) for _ in range(NBUF)]
            + [pltpu.SemaphoreType.DMA for _ in range(2 * NBUF)]
        ),
    )
    return run(idx, weight)


@jax.jit
def _full_call(idx, weight):
    acc = jnp.pad(
        _emb_call(idx[0], weight),
        ((0, BATCH - SPLIT_BATCH), (0, 0), (0, 0)))
    for s in range(1, NSPLIT):
        acc = lax.dynamic_update_slice(
            acc, _emb_call(idx[s], weight), (s * SPLIT_BATCH, 0, 0))
    return acc


def kernel(input, weight):
    idx = input.astype(jnp.int32).reshape(
        NSPLIT, NUM_WORKERS, BATCH_PER_WORKER, HIST)
    return _full_call(idx, weight)


# speedup vs baseline: 1.6757x; 1.6757x over previous
import jax
import jax.numpy as jnp
from jax import lax
from jax.experimental import pallas as pl
from jax.experimental.pallas import tpu as pltpu
from jax.experimental.pallas import tpu_sc as plsc

VOCAB = 100000
EMB_DIM = 128
BATCH = 4096
HIST = 50

NUM_CORES = 2
NUM_SUBCORES = 16
NUM_WORKERS = NUM_CORES * NUM_SUBCORES
BATCH_PER_WORKER = BATCH // NUM_WORKERS
SUPER = 2
NSUP = BATCH_PER_WORKER // SUPER
NBUF = 8
LOOK = NBUF - 1


def _emb_body(idx_hbm, table_hbm, out_hbm, idx_v, *bufs_and_sems):
    bufs = bufs_and_sems[:NBUF]
    gsems = bufs_and_sems[NBUF:2 * NBUF]
    ssems = bufs_and_sems[2 * NBUF:3 * NBUF]

    wid = lax.axis_index("s") * NUM_CORES + lax.axis_index("c")
    pltpu.sync_copy(idx_hbm.at[wid], idx_v)
    base = wid * BATCH_PER_WORKER

    def fire_gather(c, b):
        for k in range(SUPER):
            pltpu.async_copy(
                table_hbm.at[idx_v.at[c * SUPER + k]],
                bufs[b].at[k], gsems[b])

    def wait_gather(c, b):
        for k in range(SUPER):
            pltpu.make_async_copy(
                table_hbm.at[idx_v.at[0]], bufs[b].at[k], gsems[b]).wait()

    def fire_scatter(c, b):
        pltpu.async_copy(
            bufs[b], out_hbm.at[pl.ds(base + c * SUPER, SUPER)], ssems[b])

    def wait_scatter(b):
        pltpu.make_async_copy(
            bufs[b], out_hbm.at[pl.ds(base, SUPER)], ssems[b]).wait()

    for b in range(LOOK):
        fire_gather(b, b)

    wait_gather(0, 0)
    fire_scatter(0, 0)
    fire_gather(LOOK, LOOK % NBUF)

    def step(c, b, tb):
        wait_scatter(tb)
        fire_gather(c + LOOK, tb)
        wait_gather(c, b)
        fire_scatter(c, b)

    nsteady = NSUP - LOOK - 1
    ngroups = nsteady // NBUF
    nrem = nsteady % NBUF

    def outer(g, carry):
        for bp in range(NBUF):
            c = g * NBUF + 1 + bp
            step(c, (bp + 1) % NBUF, (1 + bp + LOOK) % NBUF)
        return carry

    if ngroups > 0:
        lax.fori_loop(0, ngroups, outer, 0)
    for r in range(nrem):
        c = ngroups * NBUF + 1 + r
        step(c, c % NBUF, (c + LOOK) % NBUF)

    for c in range(NSUP - LOOK, NSUP):
        b = c % NBUF
        wait_gather(c, b)
        fire_scatter(c, b)
    for b in range(NBUF):
        wait_scatter(b)


@jax.jit
def _emb_call(idx, weight):
    mesh = plsc.VectorSubcoreMesh(
        core_axis_name="c", subcore_axis_name="s",
        num_cores=NUM_CORES, num_subcores=NUM_SUBCORES,
    )
    run = pl.kernel(
        _emb_body,
        out_type=jax.ShapeDtypeStruct((BATCH, HIST, EMB_DIM), jnp.float32),
        mesh=mesh,
        scratch_types=(
            [pltpu.VMEM((BATCH_PER_WORKER, HIST), jnp.int32)]
            + [pltpu.VMEM((SUPER, HIST, EMB_DIM), jnp.float32) for _ in range(NBUF)]
            + [pltpu.SemaphoreType.DMA for _ in range(2 * NBUF)]
        ),
    )
    return run(idx, weight)


def kernel(input, weight):
    idx = input.astype(jnp.int32).reshape(NUM_WORKERS, BATCH_PER_WORKER, HIST)
    return _emb_call(idx, weight)
